# baseline (speedup 1.0000x reference)
import jax
import jax.numpy as jnp
from jax.experimental import pallas as pl


def _body(tab_ref, w1_ref, b1_ref, w2_ref, b2_ref, out_ref):
    prompt = tab_ref[:, :]
    h = jnp.tanh(
        jnp.dot(prompt, w1_ref[:, :], preferred_element_type=jnp.float32)
        + b1_ref[:].reshape(1, -1)
    )
    res = (jnp.dot(h, w2_ref[:, :], preferred_element_type=jnp.float32)
           + b2_ref[:].reshape(1, -1))
    pad = jnp.zeros((out_ref.shape[0] - res.shape[0], res.shape[1]),
                    jnp.float32)
    out_ref[:, :] = jnp.concatenate([res, pad], axis=0)


def kernel(tokens, batch_size, pre_prompt, embd_table, W1, b1, W2, b2):
    B = tokens.shape[0]
    P = pre_prompt.shape[0]
    D, H = W1.shape
    padded_rows = (P + 7) // 8 * 8
    res24 = pl.pallas_call(
        _body,
        out_shape=jax.ShapeDtypeStruct((padded_rows, D), jnp.float32),
    )(embd_table, W1, b1, W2, b2)
    return jnp.broadcast_to(res24[None, :P, :], (B, P, D))
